# baseline (speedup 1.0000x reference)
import functools

import jax
import jax.numpy as jnp
from jax import lax
from jax.experimental import pallas as pl
from jax.experimental.pallas import tpu as pltpu
from jax.experimental.pallas import tpu_sc as plsc

D_MODEL = 512
_NC = 2
_NS = 16
_NW = _NC * _NS
_B = 1024 * 200
_BPW = _B // _NW
_C = 80
_G = _BPW // _C
_NBUF = 2

_mesh = plsc.VectorSubcoreMesh(core_axis_name="c", subcore_axis_name="s")


@functools.partial(
    pl.kernel,
    out_type=jax.ShapeDtypeStruct((_NW, _G, _C, D_MODEL), jnp.float32),
    mesh=_mesh,
    scratch_types=[
        pltpu.VMEM((_G, _C), jnp.int32),
        [pltpu.VMEM((_C, D_MODEL), jnp.float32) for _ in range(_NBUF)],
        [pltpu.SemaphoreType.DMA for _ in range(_NBUF)],
        [pltpu.SemaphoreType.DMA for _ in range(_NBUF)],
    ],
)
def _emb_lookup(table_hbm, idx_hbm, out_hbm, idx_v, rows, gsem, ssem):
    wid = lax.axis_index("s") * _NC + lax.axis_index("c")
    pltpu.sync_copy(idx_hbm.at[wid], idx_v)

    def gstart(b, c):
        pltpu.async_copy(table_hbm.at[idx_v.at[c]], rows[b], gsem[b])

    def gwait(b, c):
        pltpu.make_async_copy(table_hbm.at[idx_v.at[c]], rows[b], gsem[b]).wait()

    def sstart(b, c):
        pltpu.async_copy(rows[b], out_hbm.at[wid, c], ssem[b])

    def swait(b, c):
        pltpu.make_async_copy(rows[b], out_hbm.at[wid, c], ssem[b]).wait()

    for b in range(_NBUF):
        gstart(b, b)

    def steady(g):
        for b in range(_NBUF):
            gwait(b, g + b)
            sstart(b, g + b)
        for b in range(_NBUF):
            swait(b, g + b)
            gstart(b, g + b + _NBUF)

    pl.loop(0, _G - _NBUF, step=_NBUF)(steady)

    for b in range(_NBUF):
        gwait(b, _G - _NBUF + b)
        sstart(b, _G - _NBUF + b)
    for b in range(_NBUF):
        swait(b, _G - _NBUF + b)


def kernel(x, embedding):
    idx = x.astype(jnp.int32).reshape(_NW, _G, _C)
    out = _emb_lookup(embedding, idx)
    return out.reshape(x.shape + (D_MODEL,))


# speedup vs baseline: 1.0113x; 1.0113x over previous
import functools

import jax
import jax.numpy as jnp
from jax import lax
from jax.experimental import pallas as pl
from jax.experimental.pallas import tpu as pltpu
from jax.experimental.pallas import tpu_sc as plsc

D_MODEL = 512
_NC = 2
_NS = 16
_NW = _NC * _NS
_B = 1024 * 200
_BPW = _B // _NW
_C = 40
_G = _BPW // _C
_NBUF = 4

_mesh = plsc.VectorSubcoreMesh(core_axis_name="c", subcore_axis_name="s")


@functools.partial(
    pl.kernel,
    out_type=jax.ShapeDtypeStruct((_NW, _G, _C, D_MODEL), jnp.float32),
    mesh=_mesh,
    scratch_types=[
        pltpu.VMEM((_G, _C), jnp.int32),
        [pltpu.VMEM((_C, D_MODEL), jnp.float32) for _ in range(_NBUF)],
        [pltpu.SemaphoreType.DMA for _ in range(_NBUF)],
        [pltpu.SemaphoreType.DMA for _ in range(_NBUF)],
    ],
)
def _emb_lookup(table_hbm, idx_hbm, out_hbm, idx_v, rows, gsem, ssem):
    wid = lax.axis_index("s") * _NC + lax.axis_index("c")
    pltpu.sync_copy(idx_hbm.at[wid], idx_v)

    def gstart(b, c):
        pltpu.async_copy(table_hbm.at[idx_v.at[c]], rows[b], gsem[b])

    def gwait(b, c):
        pltpu.make_async_copy(table_hbm.at[idx_v.at[c]], rows[b], gsem[b]).wait()

    def sstart(b, c):
        pltpu.async_copy(rows[b], out_hbm.at[wid, c], ssem[b])

    def swait(b, c):
        pltpu.make_async_copy(rows[b], out_hbm.at[wid, c], ssem[b]).wait()

    def step(b, j, first=False, last=False):
        pb = (b - 1) % _NBUF
        gwait(b, j + b)
        sstart(b, j + b)
        if not (first and b == 0):
            swait(pb, j + b - 1)
        if not last:
            gstart(pb, j + b - 1 + _NBUF)
        elif b == 0:
            gstart(pb, j + _NBUF - 1)

    for b in range(_NBUF - 1):
        gstart(b, b)

    for b in range(_NBUF):
        step(b, 0, first=True)

    def steady(j):
        for b in range(_NBUF):
            step(b, j)

    pl.loop(_NBUF, _G - _NBUF, step=_NBUF)(steady)

    for b in range(_NBUF):
        step(b, _G - _NBUF, last=True)
    swait(_NBUF - 1, _G - 1)


def kernel(x, embedding):
    idx = x.astype(jnp.int32).reshape(_NW, _G, _C)
    out = _emb_lookup(embedding, idx)
    return out.reshape(x.shape + (D_MODEL,))


# speedup vs baseline: 1.0129x; 1.0016x over previous
import functools

import jax
import jax.numpy as jnp
from jax import lax
from jax.experimental import pallas as pl
from jax.experimental.pallas import tpu as pltpu
from jax.experimental.pallas import tpu_sc as plsc

D_MODEL = 512
_NC = 2
_NS = 16
_NW = _NC * _NS
_B = 1024 * 200
_BPW = _B // _NW
_C = 40
_G = _BPW // _C
_NBUF = 5

_mesh = plsc.VectorSubcoreMesh(core_axis_name="c", subcore_axis_name="s")


@functools.partial(
    pl.kernel,
    out_type=jax.ShapeDtypeStruct((_NW, _G, _C, D_MODEL), jnp.float32),
    mesh=_mesh,
    scratch_types=[
        pltpu.VMEM((_G, _C), jnp.int32),
        [pltpu.VMEM((_C, D_MODEL), jnp.float32) for _ in range(_NBUF)],
        [pltpu.SemaphoreType.DMA for _ in range(_NBUF)],
        [pltpu.SemaphoreType.DMA for _ in range(_NBUF)],
    ],
)
def _emb_lookup(table_hbm, idx_hbm, out_hbm, idx_v, rows, gsem, ssem):
    wid = lax.axis_index("s") * _NC + lax.axis_index("c")
    pltpu.sync_copy(idx_hbm.at[wid], idx_v)

    def gstart(b, c):
        pltpu.async_copy(table_hbm.at[idx_v.at[c]], rows[b], gsem[b])

    def gwait(b, c):
        pltpu.make_async_copy(table_hbm.at[idx_v.at[c]], rows[b], gsem[b]).wait()

    def sstart(b, c):
        pltpu.async_copy(rows[b], out_hbm.at[wid, c], ssem[b])

    def swait(b, c):
        pltpu.make_async_copy(rows[b], out_hbm.at[wid, c], ssem[b]).wait()

    def step(b, j, first=False, last=False):
        pb = (b - 1) % _NBUF
        gwait(b, j + b)
        sstart(b, j + b)
        if not (first and b == 0):
            swait(pb, j + b - 1)
        if not last:
            gstart(pb, j + b - 1 + _NBUF)
        elif b == 0:
            gstart(pb, j + _NBUF - 1)

    for b in range(_NBUF - 1):
        gstart(b, b)

    for b in range(_NBUF):
        step(b, 0, first=True)

    def steady(j):
        for b in range(_NBUF):
            step(b, j)

    pl.loop(_NBUF, _G - _NBUF, step=_NBUF)(steady)

    for b in range(_NBUF):
        step(b, _G - _NBUF, last=True)
    swait(_NBUF - 1, _G - 1)


def kernel(x, embedding):
    idx = x.astype(jnp.int32).reshape(_NW, _G, _C)
    out = _emb_lookup(embedding, idx)
    return out.reshape(x.shape + (D_MODEL,))
